# baseline (speedup 1.0000x reference)
import functools

import jax
import jax.numpy as jnp
from jax import lax
from jax.experimental import pallas as pl
from jax.experimental.pallas import tpu as pltpu
from jax.experimental.pallas import tpu_sc as plsc

_NUM_CORES = 2
_NUM_SUBCORES = 16
_NUM_WORKERS = _NUM_CORES * _NUM_SUBCORES
_CHUNK = 128
_ROW_BLK = 1024


def _sc_mesh():
    return plsc.VectorSubcoreMesh(core_axis_name="c", subcore_axis_name="s")


def _make_deg_kernel(np_, ch):
    rows_per_tile = np_ // _NUM_SUBCORES
    n_zero = rows_per_tile // _CHUNK

    @functools.partial(
        pl.kernel,
        out_type=jax.ShapeDtypeStruct((_NUM_CORES, np_), jnp.float32),
        mesh=_sc_mesh(),
        compiler_params=pltpu.CompilerParams(needs_layout_passes=False),
        scratch_types=[
            pltpu.VMEM((ch, _CHUNK), jnp.int32),
            pltpu.VMEM((ch, _CHUNK), jnp.float32),
            pltpu.VMEM((_CHUNK,), jnp.float32),
            pltpu.VMEM_SHARED((np_,), jnp.float32),
            pltpu.SemaphoreType.DMA,
        ],
    )
    def deg_kernel(dst_hbm, w_hbm, out_hbm, dst_v, w_v, z_v, deg_sh, sem):
        cid = lax.axis_index("c")
        sid = lax.axis_index("s")
        wid = cid * _NUM_SUBCORES + sid
        for j in range(_CHUNK // 16):
            z_v[pl.ds(j * 16, 16)] = jnp.zeros((16,), jnp.float32)
        for k in range(n_zero):
            pltpu.sync_copy(
                z_v, deg_sh.at[pl.ds(sid * rows_per_tile + k * _CHUNK, _CHUNK)]
            )
        plsc.subcore_barrier()
        pltpu.sync_copy(dst_hbm.at[pl.ds(wid * ch, ch), :], dst_v)
        pltpu.sync_copy(w_hbm.at[pl.ds(wid * ch, ch), :], w_v)

        pltpu.async_copy(w_v.at[0], deg_sh.at[dst_v.at[0]], sem, add=True)

        def body(c, _):
            pltpu.async_copy(w_v.at[c], deg_sh.at[dst_v.at[c]], sem, add=True)
            pltpu.make_async_copy(w_v.at[0], deg_sh.at[dst_v.at[0]], sem).wait()
            return ()

        lax.fori_loop(1, ch, body, ())
        pltpu.make_async_copy(w_v.at[0], deg_sh.at[dst_v.at[0]], sem).wait()
        plsc.subcore_barrier()
        pltpu.sync_copy(
            deg_sh.at[pl.ds(sid * rows_per_tile, rows_per_tile)],
            out_hbm.at[cid, pl.ds(sid * rows_per_tile, rows_per_tile)],
        )

    return deg_kernel


def _make_agg_kernel(np_, d, ch, stages=5):
    rows_per_tile = np_ // _NUM_SUBCORES
    n_zero = rows_per_tile // _CHUNK
    n_col = d // 16
    sc_ch = ch // stages
    pairs = sc_ch // 2

    @functools.partial(
        pl.kernel,
        out_type=jax.ShapeDtypeStruct((_NUM_CORES, np_, d), jnp.float32),
        mesh=_sc_mesh(),
        compiler_params=pltpu.CompilerParams(needs_layout_passes=False),
        scratch_types=[
            pltpu.VMEM((sc_ch, _CHUNK), jnp.int32),
            pltpu.VMEM((sc_ch, _CHUNK), jnp.int32),
            pltpu.VMEM((sc_ch, _CHUNK), jnp.float32),
            pltpu.VMEM((_CHUNK,), jnp.float32),
            pltpu.VMEM((_CHUNK,), jnp.float32),
            pltpu.VMEM((_CHUNK, d), jnp.float32),
            pltpu.VMEM((_CHUNK, d), jnp.float32),
            pltpu.VMEM_SHARED((np_, d), jnp.float32),
            pltpu.SemaphoreType.DMA,
            pltpu.SemaphoreType.DMA,
            pltpu.SemaphoreType.DMA,
            pltpu.SemaphoreType.DMA,
            pltpu.SemaphoreType.DMA,
            pltpu.SemaphoreType.DMA,
        ],
    )
    def agg_kernel(dinv_hbm, src_hbm, dst_hbm, w_hbm, h_hbm, out_hbm,
                   src_v, dst_v, w_v, dsA, dsB, rowsA, rowsB, acc_sh,
                   gA, gB, dA, dB, sA, sB):
        cid = lax.axis_index("c")
        sid = lax.axis_index("s")
        wid = cid * _NUM_SUBCORES + sid

        def zbody(r, _):
            for j in range(n_col):
                rowsA[r, pl.ds(j * 16, 16)] = jnp.zeros((16,), jnp.float32)
            return ()

        lax.fori_loop(0, _CHUNK, zbody, ())
        for k in range(n_zero):
            pltpu.sync_copy(
                rowsA,
                acc_sh.at[pl.ds(sid * rows_per_tile + k * _CHUNK, _CHUNK), :],
            )
        plsc.subcore_barrier()

        def g_issue(rows, dsv, semg, semd, c):
            pltpu.async_copy(h_hbm.at[src_v.at[c]], rows, semg)
            pltpu.async_copy(dinv_hbm.at[src_v.at[c]], dsv, semd)

        def g_wait(rows, dsv, semg, semd):
            pltpu.make_async_copy(h_hbm.at[src_v.at[0]], rows, semg).wait()
            pltpu.make_async_copy(dinv_hbm.at[src_v.at[0]], dsv, semd).wait()

        def process(rows, dsv, semg, semd, c):
            pltpu.make_async_copy(dinv_hbm.at[src_v.at[0]], dsv, semd).wait()
            for k in range(_CHUNK // 16):
                dsv[pl.ds(k * 16, 16)] = (
                    dsv[pl.ds(k * 16, 16)] * w_v[c, pl.ds(k * 16, 16)]
                )
            pltpu.make_async_copy(h_hbm.at[src_v.at[0]], rows, semg).wait()

            def sbody(k, _):
                sv = dsv[pl.ds(k * 16, 16)]
                for l in range(16):
                    s = sv[jnp.full((16,), l, jnp.int32)]
                    e = k * 16 + l
                    for j in range(n_col):
                        rows[e, pl.ds(j * 16, 16)] = (
                            rows[e, pl.ds(j * 16, 16)] * s
                        )
                return ()

            lax.fori_loop(0, _CHUNK // 16, sbody, ())

        for st in range(stages):
            base = wid * ch + st * sc_ch
            pltpu.sync_copy(src_hbm.at[pl.ds(base, sc_ch), :], src_v)
            pltpu.sync_copy(dst_hbm.at[pl.ds(base, sc_ch), :], dst_v)
            pltpu.sync_copy(w_hbm.at[pl.ds(base, sc_ch), :], w_v)
            g_issue(rowsA, dsA, gA, dA, 0)
            g_issue(rowsB, dsB, gB, dB, 1)

            def pair(i, _):
                cA = 2 * i
                cB = 2 * i + 1
                process(rowsA, dsA, gA, dA, cA)
                cpA = pltpu.async_copy(rowsA, acc_sh.at[dst_v.at[cA]], sA,
                                       add=True)
                process(rowsB, dsB, gB, dB, cB)
                cpB = pltpu.async_copy(rowsB, acc_sh.at[dst_v.at[cB]], sB,
                                       add=True)
                nA = jnp.minimum(cA + 2, sc_ch - 1)
                nB = jnp.minimum(cB + 2, sc_ch - 1)
                cpA.wait()
                g_issue(rowsA, dsA, gA, dA, nA)
                cpB.wait()
                g_issue(rowsB, dsB, gB, dB, nB)
                return ()

            lax.fori_loop(0, pairs, pair, ())
            g_wait(rowsA, dsA, gA, dA)
            g_wait(rowsB, dsB, gB, dB)

        plsc.subcore_barrier()
        for k in range(n_zero):
            sl = pl.ds(sid * rows_per_tile + k * _CHUNK, _CHUNK)
            pltpu.sync_copy(acc_sh.at[sl, :], out_hbm.at[cid, sl, :])

    return agg_kernel


def _mm_body(x_ref, w_ref, degp_ref, h_ref, dinv_ref):
    h_ref[...] = jnp.dot(x_ref[...], w_ref[...],
                         preferred_element_type=jnp.float32)
    d = degp_ref[0, :] + degp_ref[1, :] + 1.0
    dinv_ref[...] = jnp.where(d > 0, lax.rsqrt(jnp.maximum(d, 1e-12)), 0.0)


def _make_final_body(n_blocks, g):
    def final_body(acc_ref, h_ref, dinv_ref, batch_ref, b_ref,
                   h2_ref, emb_ref, sums_sc, cnt_sc):
        i = pl.program_id(0)

        @pl.when(i == 0)
        def _():
            sums_sc[...] = jnp.zeros_like(sums_sc)
            cnt_sc[...] = jnp.zeros_like(cnt_sc)

        dinv = dinv_ref[...]
        acc = acc_ref[0] + acc_ref[1]
        h = h_ref[...]
        agg = dinv * (acc + dinv * h) + b_ref[...]
        h2 = jnp.maximum(agg, 0.0)
        h2_ref[...] = h2
        onehot = (batch_ref[...] == lax.broadcasted_iota(
            jnp.int32, (1, g), 1)).astype(jnp.float32)
        dn = (((0,), (0,)), ((), ()))
        sums_sc[...] += lax.dot_general(onehot, h2, dn,
                                        preferred_element_type=jnp.float32)
        cnt_sc[...] += lax.dot_general(
            onehot, jnp.ones_like(h2), dn, preferred_element_type=jnp.float32)

        @pl.when(i == n_blocks - 1)
        def _():
            emb_ref[...] = sums_sc[...] / jnp.maximum(cnt_sc[...], 1.0)

    return final_body


def kernel(x, edge_index, edge_weight, batch, W, b):
    n, d = x.shape
    h_dim = W.shape[1]
    e = edge_weight.shape[0]
    g = 64

    np_ = ((n + _ROW_BLK - 1) // _ROW_BLK) * _ROW_BLK
    per_w = _NUM_WORKERS * _CHUNK
    ch = (e + per_w - 1) // per_w
    ch = ((ch + 7) // 8) * 8
    ep = ch * per_w
    pad_e = ep - e

    src = edge_index[0]
    dst = edge_index[1]
    pidx = jnp.arange(pad_e, dtype=jnp.int32)
    srcp = jnp.concatenate([src, (pidx * 97) % n]).reshape(ep // _CHUNK, _CHUNK)
    dstp = jnp.concatenate([dst, (pidx * 101) % np_]).reshape(ep // _CHUNK, _CHUNK)
    wp = jnp.concatenate(
        [edge_weight, jnp.zeros((pad_e,), jnp.float32)]
    ).reshape(ep // _CHUNK, _CHUNK)
    xp = jnp.concatenate([x, jnp.zeros((np_ - n, d), x.dtype)])
    batchp = jnp.concatenate(
        [batch, jnp.full((np_ - n,), g, jnp.int32)]
    ).reshape(np_, 1)

    deg_parts = _make_deg_kernel(np_, ch)(dstp, wp)

    n_blocks = np_ // _ROW_BLK
    h_pad, dinv = pl.pallas_call(
        _mm_body,
        grid=(n_blocks,),
        in_specs=[
            pl.BlockSpec((_ROW_BLK, d), lambda i: (i, 0)),
            pl.BlockSpec((d, h_dim), lambda i: (0, 0)),
            pl.BlockSpec((_NUM_CORES, _ROW_BLK), lambda i: (0, i)),
        ],
        out_specs=[
            pl.BlockSpec((_ROW_BLK, h_dim), lambda i: (i, 0)),
            pl.BlockSpec((_ROW_BLK,), lambda i: (i,)),
        ],
        out_shape=[
            jax.ShapeDtypeStruct((np_, h_dim), jnp.float32),
            jax.ShapeDtypeStruct((np_,), jnp.float32),
        ],
    )(xp, W, deg_parts)

    acc_parts = _make_agg_kernel(np_, h_dim, ch)(dinv, srcp, dstp, wp, h_pad)

    h2_pad, emb = pl.pallas_call(
        _make_final_body(n_blocks, g),
        grid=(n_blocks,),
        in_specs=[
            pl.BlockSpec((_NUM_CORES, _ROW_BLK, h_dim), lambda i: (0, i, 0)),
            pl.BlockSpec((_ROW_BLK, h_dim), lambda i: (i, 0)),
            pl.BlockSpec((_ROW_BLK, 1), lambda i: (i, 0)),
            pl.BlockSpec((_ROW_BLK, 1), lambda i: (i, 0)),
            pl.BlockSpec((1, h_dim), lambda i: (0, 0)),
        ],
        out_specs=[
            pl.BlockSpec((_ROW_BLK, h_dim), lambda i: (i, 0)),
            pl.BlockSpec((g, h_dim), lambda i: (0, 0)),
        ],
        out_shape=[
            jax.ShapeDtypeStruct((np_, h_dim), jnp.float32),
            jax.ShapeDtypeStruct((g, h_dim), jnp.float32),
        ],
        scratch_shapes=[
            pltpu.VMEM((g, h_dim), jnp.float32),
            pltpu.VMEM((g, h_dim), jnp.float32),
        ],
    )(acc_parts, h_pad, dinv.reshape(np_, 1), batchp, b.reshape(1, h_dim))

    return (h2_pad[:n], emb)


# speedup vs baseline: 1.0225x; 1.0225x over previous
import functools

import jax
import jax.numpy as jnp
from jax import lax
from jax.experimental import pallas as pl
from jax.experimental.pallas import tpu as pltpu
from jax.experimental.pallas import tpu_sc as plsc

_NUM_CORES = 2
_NUM_SUBCORES = 16
_NUM_WORKERS = _NUM_CORES * _NUM_SUBCORES
_CHUNK = 128
_ROW_BLK = 1024


def _sc_mesh():
    return plsc.VectorSubcoreMesh(core_axis_name="c", subcore_axis_name="s")


def _make_deg_kernel(np_, ch):
    rows_per_tile = np_ // _NUM_SUBCORES
    n_zero = rows_per_tile // _CHUNK

    @functools.partial(
        pl.kernel,
        out_type=jax.ShapeDtypeStruct((_NUM_CORES, np_), jnp.float32),
        mesh=_sc_mesh(),
        compiler_params=pltpu.CompilerParams(needs_layout_passes=False),
        scratch_types=[
            pltpu.VMEM((ch, _CHUNK), jnp.int32),
            pltpu.VMEM((ch, _CHUNK), jnp.float32),
            pltpu.VMEM((_CHUNK,), jnp.float32),
            pltpu.VMEM_SHARED((np_,), jnp.float32),
            pltpu.SemaphoreType.DMA,
        ],
    )
    def deg_kernel(dst_hbm, w_hbm, out_hbm, dst_v, w_v, z_v, deg_sh, sem):
        cid = lax.axis_index("c")
        sid = lax.axis_index("s")
        wid = cid * _NUM_SUBCORES + sid
        for j in range(_CHUNK // 16):
            z_v[pl.ds(j * 16, 16)] = jnp.zeros((16,), jnp.float32)
        for k in range(n_zero):
            pltpu.sync_copy(
                z_v, deg_sh.at[pl.ds(sid * rows_per_tile + k * _CHUNK, _CHUNK)]
            )
        plsc.subcore_barrier()
        pltpu.sync_copy(dst_hbm.at[pl.ds(wid * ch, ch), :], dst_v)
        pltpu.sync_copy(w_hbm.at[pl.ds(wid * ch, ch), :], w_v)

        pltpu.async_copy(w_v.at[0], deg_sh.at[dst_v.at[0]], sem, add=True)

        def body(c, _):
            pltpu.async_copy(w_v.at[c], deg_sh.at[dst_v.at[c]], sem, add=True)
            pltpu.make_async_copy(w_v.at[0], deg_sh.at[dst_v.at[0]], sem).wait()
            return ()

        lax.fori_loop(1, ch, body, ())
        pltpu.make_async_copy(w_v.at[0], deg_sh.at[dst_v.at[0]], sem).wait()
        plsc.subcore_barrier()
        pltpu.sync_copy(
            deg_sh.at[pl.ds(sid * rows_per_tile, rows_per_tile)],
            out_hbm.at[cid, pl.ds(sid * rows_per_tile, rows_per_tile)],
        )

    return deg_kernel


def _make_agg_kernel(np_, d, ch, stages=5):
    rows_per_tile = np_ // _NUM_SUBCORES
    n_zero = rows_per_tile // _CHUNK
    n_col = d // 16
    sc_ch = ch // stages
    pairs = sc_ch // 2

    @functools.partial(
        pl.kernel,
        out_type=jax.ShapeDtypeStruct((_NUM_CORES, np_, d), jnp.float32),
        mesh=_sc_mesh(),
        compiler_params=pltpu.CompilerParams(needs_layout_passes=False),
        scratch_types=[
            pltpu.VMEM((sc_ch, _CHUNK), jnp.int32),
            pltpu.VMEM((sc_ch, _CHUNK), jnp.int32),
            pltpu.VMEM((sc_ch, _CHUNK), jnp.float32),
            pltpu.VMEM((_CHUNK,), jnp.float32),
            pltpu.VMEM((_CHUNK,), jnp.float32),
            pltpu.VMEM((_CHUNK, d), jnp.float32),
            pltpu.VMEM((_CHUNK, d), jnp.float32),
            pltpu.VMEM_SHARED((np_, d), jnp.float32),
            pltpu.SemaphoreType.DMA,
            pltpu.SemaphoreType.DMA,
            pltpu.SemaphoreType.DMA,
            pltpu.SemaphoreType.DMA,
            pltpu.SemaphoreType.DMA,
            pltpu.SemaphoreType.DMA,
        ],
    )
    def agg_kernel(dinv_hbm, src_hbm, dst_hbm, w_hbm, h_hbm, out_hbm,
                   src_v, dst_v, w_v, dsA, dsB, rowsA, rowsB, acc_sh,
                   gA, gB, dA, dB, sA, sB):
        cid = lax.axis_index("c")
        sid = lax.axis_index("s")
        wid = cid * _NUM_SUBCORES + sid

        def zbody(r, _):
            for j in range(n_col):
                rowsA[r, pl.ds(j * 16, 16)] = jnp.zeros((16,), jnp.float32)
            return ()

        lax.fori_loop(0, _CHUNK, zbody, ())
        for k in range(n_zero):
            pltpu.sync_copy(
                rowsA,
                acc_sh.at[pl.ds(sid * rows_per_tile + k * _CHUNK, _CHUNK), :],
            )
        plsc.subcore_barrier()

        def g_issue(rows, dsv, semg, semd, c):
            pltpu.async_copy(h_hbm.at[src_v.at[c]], rows, semg)
            pltpu.async_copy(dinv_hbm.at[src_v.at[c]], dsv, semd)

        def g_wait(rows, dsv, semg, semd):
            pltpu.make_async_copy(h_hbm.at[src_v.at[0]], rows, semg).wait()
            pltpu.make_async_copy(dinv_hbm.at[src_v.at[0]], dsv, semd).wait()

        def process(rows, dsv, semg, semd, c):
            pltpu.make_async_copy(dinv_hbm.at[src_v.at[0]], dsv, semd).wait()
            for k in range(_CHUNK // 16):
                dsv[pl.ds(k * 16, 16)] = (
                    dsv[pl.ds(k * 16, 16)] * w_v[c, pl.ds(k * 16, 16)]
                )
            pltpu.make_async_copy(h_hbm.at[src_v.at[0]], rows, semg).wait()

            def sbody(k, _):
                sv = dsv[pl.ds(k * 16, 16)]
                for l in range(16):
                    s = sv[jnp.full((16,), l, jnp.int32)]
                    e = k * 16 + l
                    for j in range(n_col):
                        rows[e, pl.ds(j * 16, 16)] = (
                            rows[e, pl.ds(j * 16, 16)] * s
                        )
                return ()

            lax.fori_loop(0, _CHUNK // 16, sbody, ())

        for st in range(stages):
            base = wid * ch + st * sc_ch
            pltpu.sync_copy(src_hbm.at[pl.ds(base, sc_ch), :], src_v)
            pltpu.sync_copy(dst_hbm.at[pl.ds(base, sc_ch), :], dst_v)
            pltpu.sync_copy(w_hbm.at[pl.ds(base, sc_ch), :], w_v)
            g_issue(rowsA, dsA, gA, dA, 0)
            g_issue(rowsB, dsB, gB, dB, 1)

            def pair(i, _):
                cA = 2 * i
                cB = 2 * i + 1
                process(rowsA, dsA, gA, dA, cA)
                cpA = pltpu.async_copy(rowsA, acc_sh.at[dst_v.at[cA]], sA,
                                       add=True)
                process(rowsB, dsB, gB, dB, cB)
                cpB = pltpu.async_copy(rowsB, acc_sh.at[dst_v.at[cB]], sB,
                                       add=True)
                nA = jnp.minimum(cA + 2, sc_ch - 1)
                nB = jnp.minimum(cB + 2, sc_ch - 1)
                cpA.wait()
                g_issue(rowsA, dsA, gA, dA, nA)
                cpB.wait()
                g_issue(rowsB, dsB, gB, dB, nB)
                return ()

            lax.fori_loop(0, pairs, pair, ())
            g_wait(rowsA, dsA, gA, dA)
            g_wait(rowsB, dsB, gB, dB)

        plsc.subcore_barrier()
        for k in range(n_zero):
            sl = pl.ds(sid * rows_per_tile + k * _CHUNK, _CHUNK)
            pltpu.sync_copy(acc_sh.at[sl, :], out_hbm.at[cid, sl, :])

    return agg_kernel


def _mm_body(x_ref, w_ref, h_ref):
    h_ref[...] = jnp.dot(x_ref[...], w_ref[...],
                         preferred_element_type=jnp.float32)


def _dinv_body(degp_ref, dinv_ref):
    d = degp_ref[0, :] + degp_ref[1, :] + 1.0
    dinv_ref[...] = jnp.where(d > 0, lax.rsqrt(jnp.maximum(d, 1e-12)), 0.0)


def _make_final_body(n_blocks, g):
    def final_body(acc_ref, h_ref, dinv_ref, batch_ref, b_ref,
                   h2_ref, emb_ref, sums_sc, cnt_sc):
        i = pl.program_id(0)

        @pl.when(i == 0)
        def _():
            sums_sc[...] = jnp.zeros_like(sums_sc)
            cnt_sc[...] = jnp.zeros_like(cnt_sc)

        dinv = dinv_ref[...]
        acc = acc_ref[0] + acc_ref[1]
        h = h_ref[...]
        agg = dinv * (acc + dinv * h) + b_ref[...]
        h2 = jnp.maximum(agg, 0.0)
        h2_ref[...] = h2
        onehot = (batch_ref[...] == lax.broadcasted_iota(
            jnp.int32, (1, g), 1)).astype(jnp.float32)
        dn = (((0,), (0,)), ((), ()))
        sums_sc[...] += lax.dot_general(onehot, h2, dn,
                                        preferred_element_type=jnp.float32)
        cnt_sc[...] += lax.dot_general(
            onehot, jnp.ones_like(h2), dn, preferred_element_type=jnp.float32)

        @pl.when(i == n_blocks - 1)
        def _():
            emb_ref[...] = sums_sc[...] / jnp.maximum(cnt_sc[...], 1.0)

    return final_body


def kernel(x, edge_index, edge_weight, batch, W, b):
    n, d = x.shape
    h_dim = W.shape[1]
    e = edge_weight.shape[0]
    g = 64

    np_ = ((n + _ROW_BLK - 1) // _ROW_BLK) * _ROW_BLK
    per_w = _NUM_WORKERS * _CHUNK
    ch = (e + per_w - 1) // per_w
    ch = ((ch + 7) // 8) * 8
    ep = ch * per_w
    pad_e = ep - e

    src = edge_index[0]
    dst = edge_index[1]
    pidx = jnp.arange(pad_e, dtype=jnp.int32)
    srcp = jnp.concatenate([src, (pidx * 97) % n]).reshape(ep // _CHUNK, _CHUNK)
    dstp = jnp.concatenate([dst, (pidx * 101) % np_]).reshape(ep // _CHUNK, _CHUNK)
    wp = jnp.concatenate(
        [edge_weight, jnp.zeros((pad_e,), jnp.float32)]
    ).reshape(ep // _CHUNK, _CHUNK)
    xp = jnp.concatenate([x, jnp.zeros((np_ - n, d), x.dtype)])
    batchp = jnp.concatenate(
        [batch, jnp.full((np_ - n,), g, jnp.int32)]
    ).reshape(np_, 1)

    deg_parts = _make_deg_kernel(np_, ch)(dstp, wp)

    dinv = pl.pallas_call(
        _dinv_body,
        out_shape=jax.ShapeDtypeStruct((np_,), jnp.float32),
    )(deg_parts)

    n_blocks = np_ // _ROW_BLK
    h_pad = pl.pallas_call(
        _mm_body,
        grid=(n_blocks,),
        in_specs=[
            pl.BlockSpec((_ROW_BLK, d), lambda i: (i, 0)),
            pl.BlockSpec((d, h_dim), lambda i: (0, 0)),
        ],
        out_specs=pl.BlockSpec((_ROW_BLK, h_dim), lambda i: (i, 0)),
        out_shape=jax.ShapeDtypeStruct((np_, h_dim), jnp.float32),
    )(xp, W)

    acc_parts = _make_agg_kernel(np_, h_dim, ch)(dinv, srcp, dstp, wp, h_pad)

    h2_pad, emb = pl.pallas_call(
        _make_final_body(n_blocks, g),
        grid=(n_blocks,),
        in_specs=[
            pl.BlockSpec((_NUM_CORES, _ROW_BLK, h_dim), lambda i: (0, i, 0)),
            pl.BlockSpec((_ROW_BLK, h_dim), lambda i: (i, 0)),
            pl.BlockSpec((_ROW_BLK, 1), lambda i: (i, 0)),
            pl.BlockSpec((_ROW_BLK, 1), lambda i: (i, 0)),
            pl.BlockSpec((1, h_dim), lambda i: (0, 0)),
        ],
        out_specs=[
            pl.BlockSpec((_ROW_BLK, h_dim), lambda i: (i, 0)),
            pl.BlockSpec((g, h_dim), lambda i: (0, 0)),
        ],
        out_shape=[
            jax.ShapeDtypeStruct((np_, h_dim), jnp.float32),
            jax.ShapeDtypeStruct((g, h_dim), jnp.float32),
        ],
        scratch_shapes=[
            pltpu.VMEM((g, h_dim), jnp.float32),
            pltpu.VMEM((g, h_dim), jnp.float32),
        ],
    )(acc_parts, h_pad, dinv.reshape(np_, 1), batchp, b.reshape(1, h_dim))

    return (h2_pad[:n], emb)


# speedup vs baseline: 1.1350x; 1.1100x over previous
import functools

import jax
import jax.numpy as jnp
from jax import lax
from jax.experimental import pallas as pl
from jax.experimental.pallas import tpu as pltpu
from jax.experimental.pallas import tpu_sc as plsc

_NUM_CORES = 2
_NUM_SUBCORES = 16
_NUM_WORKERS = _NUM_CORES * _NUM_SUBCORES
_CHUNK = 128
_ROW_BLK = 1024


def _sc_mesh():
    return plsc.VectorSubcoreMesh(core_axis_name="c", subcore_axis_name="s")


def _make_deg_kernel(np_, ch):
    rows_per_tile = np_ // _NUM_SUBCORES
    n_zero = rows_per_tile // _CHUNK

    @functools.partial(
        pl.kernel,
        out_type=jax.ShapeDtypeStruct((_NUM_CORES, np_), jnp.float32),
        mesh=_sc_mesh(),
        compiler_params=pltpu.CompilerParams(needs_layout_passes=False),
        scratch_types=[
            pltpu.VMEM((ch, _CHUNK), jnp.int32),
            pltpu.VMEM((ch, _CHUNK), jnp.float32),
            pltpu.VMEM((_CHUNK,), jnp.float32),
            pltpu.VMEM_SHARED((np_,), jnp.float32),
            pltpu.SemaphoreType.DMA,
        ],
    )
    def deg_kernel(dst_hbm, w_hbm, out_hbm, dst_v, w_v, z_v, deg_sh, sem):
        cid = lax.axis_index("c")
        sid = lax.axis_index("s")
        wid = cid * _NUM_SUBCORES + sid
        for j in range(_CHUNK // 16):
            z_v[pl.ds(j * 16, 16)] = jnp.zeros((16,), jnp.float32)
        for k in range(n_zero):
            pltpu.sync_copy(
                z_v, deg_sh.at[pl.ds(sid * rows_per_tile + k * _CHUNK, _CHUNK)]
            )
        plsc.subcore_barrier()
        pltpu.sync_copy(dst_hbm.at[pl.ds(wid * ch, ch), :], dst_v)
        pltpu.sync_copy(w_hbm.at[pl.ds(wid * ch, ch), :], w_v)

        pltpu.async_copy(w_v.at[0], deg_sh.at[dst_v.at[0]], sem, add=True)

        def body(c, _):
            pltpu.async_copy(w_v.at[c], deg_sh.at[dst_v.at[c]], sem, add=True)
            pltpu.make_async_copy(w_v.at[0], deg_sh.at[dst_v.at[0]], sem).wait()
            return ()

        lax.fori_loop(1, ch, body, ())
        pltpu.make_async_copy(w_v.at[0], deg_sh.at[dst_v.at[0]], sem).wait()
        plsc.subcore_barrier()
        pltpu.sync_copy(
            deg_sh.at[pl.ds(sid * rows_per_tile, rows_per_tile)],
            out_hbm.at[cid, pl.ds(sid * rows_per_tile, rows_per_tile)],
        )

    return deg_kernel


def _make_agg_kernel(np_, d, ch, stages=5):
    rows_per_tile = np_ // _NUM_SUBCORES
    n_zero = rows_per_tile // _CHUNK
    n_col = d // 16
    sc_ch = ch // stages
    pairs = sc_ch // 2

    ck = 64
    cps = sc_ch * 2

    @functools.partial(
        pl.kernel,
        out_type=jax.ShapeDtypeStruct((_NUM_CORES, np_, d), jnp.float32),
        mesh=_sc_mesh(),
        compiler_params=pltpu.CompilerParams(needs_layout_passes=False),
        scratch_types=[
            pltpu.VMEM((sc_ch, _CHUNK), jnp.int32),
            pltpu.VMEM((sc_ch, _CHUNK), jnp.float32),
        ]
        + [pltpu.VMEM((ck,), jnp.int32) for _ in range(4)]
        + [pltpu.VMEM((ck,), jnp.float32) for _ in range(4)]
        + [pltpu.VMEM((ck, d), jnp.float32) for _ in range(4)]
        + [pltpu.VMEM_SHARED((np_, d), jnp.float32)]
        + [pltpu.SemaphoreType.DMA] * 16,
    )
    def agg_kernel(dinv_hbm, src_hbm, dst_hbm, dstf_hbm, w_hbm, h_hbm,
                   out_hbm,
                   src_v, w_v,
                   db0, db1, db2, db3, ds0, ds1, ds2, ds3,
                   r0, r1, r2, r3, acc_sh,
                   g0, g1, g2, g3, e0, e1, e2, e3, s0, s1, s2, s3,
                   b0, b1, b2, b3):
        cid = lax.axis_index("c")
        sid = lax.axis_index("s")
        wid = cid * _NUM_SUBCORES + sid
        db = [db0, db1, db2, db3]
        dsv = [ds0, ds1, ds2, ds3]
        rows = [r0, r1, r2, r3]
        gsem = [g0, g1, g2, g3]
        esem = [e0, e1, e2, e3]
        ssem = [s0, s1, s2, s3]
        bsem = [b0, b1, b2, b3]

        def zbody(r, _):
            for j in range(n_col):
                r0[r, pl.ds(j * 16, 16)] = jnp.zeros((16,), jnp.float32)
            return ()

        lax.fori_loop(0, ck, zbody, ())
        for k in range(rows_per_tile // ck):
            pltpu.sync_copy(
                r0, acc_sh.at[pl.ds(sid * rows_per_tile + k * ck, ck), :]
            )
        plsc.subcore_barrier()

        def sidx(ci):
            return src_v.at[ci // 2, pl.ds((ci % 2) * ck, ck)]

        def dstcopy(b, off, ci):
            o = pl.multiple_of(off + ci * ck, 8)
            pltpu.async_copy(dstf_hbm.at[pl.ds(o, ck)], db[b], bsem[b])

        def dstwait(b):
            pltpu.make_async_copy(dstf_hbm.at[pl.ds(0, ck)], db[b],
                                  bsem[b]).wait()

        def g_issue(b, ci):
            pltpu.async_copy(h_hbm.at[sidx(ci)], rows[b], gsem[b])
            pltpu.async_copy(dinv_hbm.at[sidx(ci)], dsv[b], esem[b])

        def g_wait(b):
            pltpu.make_async_copy(h_hbm.at[sidx(0)], rows[b], gsem[b]).wait()
            pltpu.make_async_copy(dinv_hbm.at[sidx(0)], dsv[b], esem[b]).wait()

        def sc_issue(b):
            dstwait(b)
            pltpu.async_copy(rows[b], acc_sh.at[db[b]], ssem[b], add=True)

        def sc_wait(b):
            pltpu.make_async_copy(rows[b], acc_sh.at[db[b]], ssem[b]).wait()

        def process(b, ci):
            r = ci // 2
            o = (ci % 2) * ck
            pltpu.make_async_copy(dinv_hbm.at[sidx(0)], dsv[b],
                                  esem[b]).wait()
            for k in range(ck // 16):
                dsv[b][pl.ds(k * 16, 16)] = (
                    dsv[b][pl.ds(k * 16, 16)]
                    * w_v[r, pl.ds(o + k * 16, 16)]
                )
            pltpu.make_async_copy(h_hbm.at[sidx(0)], rows[b], gsem[b]).wait()

            def sbody(k, _):
                sv = dsv[b][pl.ds(k * 16, 16)]
                for l in range(16):
                    s = sv[jnp.full((16,), l, jnp.int32)]
                    e = k * 16 + l
                    for j in range(n_col):
                        rows[b][e, pl.ds(j * 16, 16)] = (
                            rows[b][e, pl.ds(j * 16, 16)] * s
                        )
                return ()

            lax.fori_loop(0, ck // 16, sbody, ())

        def stage_body(st, _):
            base = pl.multiple_of(wid * ch + st * sc_ch, 8)
            off = base * _CHUNK
            pltpu.sync_copy(src_hbm.at[pl.ds(base, sc_ch), :], src_v)
            pltpu.sync_copy(w_hbm.at[pl.ds(base, sc_ch), :], w_v)
            dstcopy(0, off, 0)
            g_issue(0, 0)
            dstcopy(1, off, 1)
            g_issue(1, 1)

            def super_body(i, _):
                for b in range(4):
                    ci = 4 * i + b
                    process(b, ci)
                    sc_issue(b)
                    b2 = (b + 2) % 4

                    @pl.when(ci >= 2)
                    def _():
                        sc_wait(b2)

                    ci2 = jnp.minimum(ci + 2, cps - 1)
                    dstcopy(b2, off, ci2)
                    g_issue(b2, ci2)
                return ()

            lax.fori_loop(0, cps // 4, super_body, ())
            sc_wait(2)
            sc_wait(3)
            g_wait(0)
            g_wait(1)
            dstwait(0)
            dstwait(1)
            return ()

        lax.fori_loop(0, stages, stage_body, ())

        plsc.subcore_barrier()
        for k in range(n_zero):
            sl = pl.ds(sid * rows_per_tile + k * _CHUNK, _CHUNK)
            pltpu.sync_copy(acc_sh.at[sl, :], out_hbm.at[cid, sl, :])

    return agg_kernel


def _mm_body(x_ref, w_ref, h_ref):
    h_ref[...] = jnp.dot(x_ref[...], w_ref[...],
                         preferred_element_type=jnp.float32)


def _dinv_body(degp_ref, dinv_ref):
    d = degp_ref[0, :] + degp_ref[1, :] + 1.0
    dinv_ref[...] = jnp.where(d > 0, lax.rsqrt(jnp.maximum(d, 1e-12)), 0.0)


def _make_final_body(n_blocks, g):
    def final_body(acc_ref, h_ref, dinv_ref, batch_ref, b_ref,
                   h2_ref, emb_ref, sums_sc, cnt_sc):
        i = pl.program_id(0)

        @pl.when(i == 0)
        def _():
            sums_sc[...] = jnp.zeros_like(sums_sc)
            cnt_sc[...] = jnp.zeros_like(cnt_sc)

        dinv = dinv_ref[...]
        acc = acc_ref[0] + acc_ref[1]
        h = h_ref[...]
        agg = dinv * (acc + dinv * h) + b_ref[...]
        h2 = jnp.maximum(agg, 0.0)
        h2_ref[...] = h2
        onehot = (batch_ref[...] == lax.broadcasted_iota(
            jnp.int32, (1, g), 1)).astype(jnp.float32)
        dn = (((0,), (0,)), ((), ()))
        sums_sc[...] += lax.dot_general(onehot, h2, dn,
                                        preferred_element_type=jnp.float32)
        cnt_sc[...] += lax.dot_general(
            onehot, jnp.ones_like(h2), dn, preferred_element_type=jnp.float32)

        @pl.when(i == n_blocks - 1)
        def _():
            emb_ref[...] = sums_sc[...] / jnp.maximum(cnt_sc[...], 1.0)

    return final_body


def kernel(x, edge_index, edge_weight, batch, W, b):
    n, d = x.shape
    h_dim = W.shape[1]
    e = edge_weight.shape[0]
    g = 64

    np_ = ((n + _ROW_BLK - 1) // _ROW_BLK) * _ROW_BLK
    per_w = _NUM_WORKERS * _CHUNK
    ch = (e + per_w - 1) // per_w
    ch = ((ch + 7) // 8) * 8
    ep = ch * per_w
    pad_e = ep - e

    src = edge_index[0]
    dst = edge_index[1]
    pidx = jnp.arange(pad_e, dtype=jnp.int32)
    srcp = jnp.concatenate([src, (pidx * 97) % n]).reshape(ep // _CHUNK, _CHUNK)
    dstp = jnp.concatenate([dst, (pidx * 101) % np_]).reshape(ep // _CHUNK, _CHUNK)
    wp = jnp.concatenate(
        [edge_weight, jnp.zeros((pad_e,), jnp.float32)]
    ).reshape(ep // _CHUNK, _CHUNK)
    xp = jnp.concatenate([x, jnp.zeros((np_ - n, d), x.dtype)])
    batchp = jnp.concatenate(
        [batch, jnp.full((np_ - n,), g, jnp.int32)]
    ).reshape(np_, 1)

    deg_parts = _make_deg_kernel(np_, ch)(dstp, wp)

    dinv = pl.pallas_call(
        _dinv_body,
        out_shape=jax.ShapeDtypeStruct((np_,), jnp.float32),
    )(deg_parts)

    n_blocks = np_ // _ROW_BLK
    h_pad = pl.pallas_call(
        _mm_body,
        grid=(n_blocks,),
        in_specs=[
            pl.BlockSpec((_ROW_BLK, d), lambda i: (i, 0)),
            pl.BlockSpec((d, h_dim), lambda i: (0, 0)),
        ],
        out_specs=pl.BlockSpec((_ROW_BLK, h_dim), lambda i: (i, 0)),
        out_shape=jax.ShapeDtypeStruct((np_, h_dim), jnp.float32),
    )(xp, W)

    acc_parts = _make_agg_kernel(np_, h_dim, ch)(dinv, srcp, dstp,
                                                 dstp.reshape(ep), wp, h_pad)

    h2_pad, emb = pl.pallas_call(
        _make_final_body(n_blocks, g),
        grid=(n_blocks,),
        in_specs=[
            pl.BlockSpec((_NUM_CORES, _ROW_BLK, h_dim), lambda i: (0, i, 0)),
            pl.BlockSpec((_ROW_BLK, h_dim), lambda i: (i, 0)),
            pl.BlockSpec((_ROW_BLK, 1), lambda i: (i, 0)),
            pl.BlockSpec((_ROW_BLK, 1), lambda i: (i, 0)),
            pl.BlockSpec((1, h_dim), lambda i: (0, 0)),
        ],
        out_specs=[
            pl.BlockSpec((_ROW_BLK, h_dim), lambda i: (i, 0)),
            pl.BlockSpec((g, h_dim), lambda i: (0, 0)),
        ],
        out_shape=[
            jax.ShapeDtypeStruct((np_, h_dim), jnp.float32),
            jax.ShapeDtypeStruct((g, h_dim), jnp.float32),
        ],
        scratch_shapes=[
            pltpu.VMEM((g, h_dim), jnp.float32),
            pltpu.VMEM((g, h_dim), jnp.float32),
        ],
    )(acc_parts, h_pad, dinv.reshape(np_, 1), batchp, b.reshape(1, h_dim))

    return (h2_pad[:n], emb)
